# baseline (speedup 1.0000x reference)
import functools

import jax
import jax.numpy as jnp
from jax import lax
from jax.experimental import pallas as pl
from jax.experimental.pallas import tpu as pltpu
from jax.experimental.pallas import tpu_sc as plsc

N = 10000
E = 320000
D = 128
G = 64

NC = 2
NS = 16
NW = NC * NS
CH = 80
NCH = 128
EP = CH * NCH
E_PAD = EP * NW
NP = NS * 640
RW = NP // NS

_MESH = plsc.VectorSubcoreMesh(core_axis_name="c", subcore_axis_name="s")


@functools.partial(
    pl.kernel,
    out_type=jax.ShapeDtypeStruct((NC, NP), jnp.float32),
    mesh=_MESH,
    scratch_types=[
        pltpu.VMEM((NCH, 2, CH), jnp.int32),
        pltpu.VMEM((CH,), jnp.float32),
        pltpu.VMEM((RW,), jnp.float32),
        pltpu.VMEM_SHARED((NP,), jnp.float32),
    ],
)
def _sc_degree(epack_hbm, out_hbm, idx_v, ones_v, zbuf_v, acc_sh):
    c = lax.axis_index("c")
    s = lax.axis_index("s")
    wid = c * NS + s
    for k in range(CH // 16):
        ones_v[pl.ds(k * 16, 16)] = jnp.ones((16,), jnp.float32)
    for k in range(RW // 16):
        zbuf_v[pl.ds(k * 16, 16)] = jnp.zeros((16,), jnp.float32)
    pltpu.sync_copy(epack_hbm.at[pl.ds(wid * NCH, NCH)], idx_v)
    pltpu.sync_copy(zbuf_v, acc_sh.at[pl.ds(s * RW, RW)])
    plsc.subcore_barrier()

    def body(i, carry):
        pltpu.sync_copy(ones_v, acc_sh.at[idx_v.at[i, 1]], add=True)
        return carry

    lax.fori_loop(0, NCH, body, 0)
    plsc.subcore_barrier()
    pltpu.sync_copy(acc_sh.at[pl.ds(s * RW, RW)], out_hbm.at[c, pl.ds(s * RW, RW)])


@functools.partial(
    pl.kernel,
    out_type=jax.ShapeDtypeStruct((NC, NP, D), jnp.float32),
    mesh=_MESH,
    scratch_types=[
        pltpu.VMEM((2, CH), jnp.int32),
        pltpu.VMEM((CH, D), jnp.float32),
        pltpu.VMEM_SHARED((NP, D), jnp.float32),
        pltpu.SemaphoreType.DMA,
    ],
)
def _sc_aggregate(h_hbm, epack_hbm, zeros_hbm, out_hbm,
                  eb, rows, acc_sh, gs):
    c = lax.axis_index("c")
    s = lax.axis_index("s")
    wid = c * NS + s
    base = wid * NCH
    pltpu.sync_copy(zeros_hbm.at[pl.ds(s * RW, RW)], acc_sh.at[pl.ds(s * RW, RW)])
    plsc.subcore_barrier()

    def body(i, carry):
        pltpu.sync_copy(epack_hbm.at[base + i], eb)
        pltpu.async_copy(h_hbm.at[eb.at[0]], rows, gs).wait()
        pltpu.sync_copy(rows, acc_sh.at[eb.at[1]], add=True)
        return carry

    lax.fori_loop(0, NCH, body, 0)
    plsc.subcore_barrier()
    pltpu.sync_copy(acc_sh.at[pl.ds(s * RW, RW)], out_hbm.at[c, pl.ds(s * RW, RW)])


def _tc_mm1(x_ref, w1_ref, h_ref):
    h_ref[...] = jnp.dot(x_ref[...], w1_ref[...],
                         preferred_element_type=jnp.float32)


def _tc1(h_ref, degpt_ref, hs_ref, dis_ref):
    deg = degpt_ref[:, 0:1] + degpt_ref[:, 1:2] + 1.0
    dis = lax.rsqrt(deg[:N, :])
    dis_ref[...] = dis
    hs_ref[...] = h_ref[...] * dis


def _tc2(p_ref, hs1_ref, dis_ref, b1_ref, w2_ref, hs2_ref):
    agg = p_ref[0] + p_ref[1]
    agg = agg[:N, :] + hs1_ref[...]
    dis = dis_ref[...]
    z = jnp.maximum(agg * dis + b1_ref[...], 0.0)
    hs2_ref[...] = jnp.dot(z, w2_ref[...], preferred_element_type=jnp.float32) * dis


def _tc3(p_ref, hs2_ref, dis_ref, b2_ref, batch_ref, wfc_ref, bfc_ref, out_ref):
    agg = p_ref[0] + p_ref[1]
    agg = agg[:N, :] + hs2_ref[...]
    z = jnp.maximum(agg * dis_ref[...] + b2_ref[...], 0.0)
    sel = (batch_ref[...] == lax.broadcasted_iota(jnp.int32, (G, 1), 0))
    sel = sel.astype(jnp.float32)
    sums = jnp.dot(sel, z, preferred_element_type=jnp.float32)
    counts = jnp.sum(sel, axis=1, keepdims=True)
    pooled = sums / jnp.maximum(counts, 1.0)
    logits = jnp.dot(pooled, wfc_ref[...], preferred_element_type=jnp.float32)
    out_ref[...] = jax.nn.sigmoid(logits + bfc_ref[...])


def kernel(x, edge_index, batch, W1, b1, W2, b2, Wfc, bfc):
    pad = E_PAD - E
    dst_pad = N + (jnp.arange(pad, dtype=jnp.int32) % (NP - N))
    src2d = jnp.concatenate(
        [edge_index[0], jnp.zeros((pad,), jnp.int32)]).reshape(E_PAD // CH, CH)
    dst2d = jnp.concatenate(
        [edge_index[1], dst_pad]).reshape(E_PAD // CH, CH)
    epack = jnp.stack([src2d, dst2d], axis=1)

    h1 = pl.pallas_call(
        _tc_mm1, out_shape=jax.ShapeDtypeStruct((N, D), jnp.float32),
    )(x, W1)

    degp = _sc_degree(epack)
    degpt = degp.T

    hs1, dis = pl.pallas_call(
        _tc1,
        out_shape=[jax.ShapeDtypeStruct((N, D), jnp.float32),
                   jax.ShapeDtypeStruct((N, 1), jnp.float32)],
    )(h1, degpt)

    zeros_nd = jnp.zeros((NP, D), jnp.float32)
    p1 = _sc_aggregate(hs1, epack, zeros_nd)

    hs2 = pl.pallas_call(
        _tc2,
        out_shape=jax.ShapeDtypeStruct((N, D), jnp.float32),
    )(p1, hs1, dis, b1, W2)

    p2 = _sc_aggregate(hs2, epack, zeros_nd)

    out = pl.pallas_call(
        _tc3,
        out_shape=jax.ShapeDtypeStruct((G, 1), jnp.float32),
    )(p2, hs2, dis, b2, batch[None], Wfc, bfc)
    return out


# speedup vs baseline: 1.8369x; 1.8369x over previous
import functools

import jax
import jax.numpy as jnp
from jax import lax
from jax.experimental import pallas as pl
from jax.experimental.pallas import tpu as pltpu
from jax.experimental.pallas import tpu_sc as plsc

N = 10000
E = 320000
D = 128
G = 64

NC = 2
NS = 16
NW = NC * NS
EW = E // NW
CH = 80
NCH = EW // CH
NP = NS * 640
RW = NP // NS

_MESH = plsc.VectorSubcoreMesh(core_axis_name="c", subcore_axis_name="s")


@functools.partial(
    pl.kernel,
    out_type=jax.ShapeDtypeStruct((NC, NP), jnp.float32),
    mesh=_MESH,
    scratch_types=[
        pltpu.VMEM((CH,), jnp.int32),
        pltpu.VMEM((CH,), jnp.float32),
        pltpu.VMEM((RW,), jnp.float32),
        pltpu.VMEM_SHARED((NP,), jnp.float32),
    ],
)
def _sc_degree(dst_hbm, out_hbm, idx_v, ones_v, zbuf_v, acc_sh):
    c = lax.axis_index("c")
    s = lax.axis_index("s")
    wid = c * NS + s
    for k in range(CH // 16):
        ones_v[pl.ds(k * 16, 16)] = jnp.ones((16,), jnp.float32)
    for k in range(RW // 16):
        zbuf_v[pl.ds(k * 16, 16)] = jnp.zeros((16,), jnp.float32)
    pltpu.sync_copy(zbuf_v, acc_sh.at[pl.ds(s * RW, RW)])
    plsc.subcore_barrier()

    def body(i, carry):
        base = wid * EW + i * CH
        pltpu.sync_copy(dst_hbm.at[pl.ds(base, CH)], idx_v)
        pltpu.sync_copy(ones_v, acc_sh.at[idx_v], add=True)
        return carry

    lax.fori_loop(0, NCH, body, 0)
    plsc.subcore_barrier()
    pltpu.sync_copy(acc_sh.at[pl.ds(s * RW, RW)], out_hbm.at[c, pl.ds(s * RW, RW)])


@functools.partial(
    pl.kernel,
    out_type=jax.ShapeDtypeStruct((NC, NP, D), jnp.float32),
    mesh=_MESH,
    scratch_types=[
        pltpu.VMEM((CH,), jnp.int32),
        pltpu.VMEM((CH,), jnp.int32),
        pltpu.VMEM((CH, D), jnp.float32),
        pltpu.VMEM_SHARED((NP, D), jnp.float32),
        pltpu.SemaphoreType.DMA,
    ],
)
def _sc_aggregate(h_hbm, src_hbm, dst_hbm, zeros_hbm, out_hbm,
                  src_v, dst_v, rows_v, acc_sh, sem):
    c = lax.axis_index("c")
    s = lax.axis_index("s")
    wid = c * NS + s
    pltpu.sync_copy(zeros_hbm.at[pl.ds(s * RW, RW)], acc_sh.at[pl.ds(s * RW, RW)])
    plsc.subcore_barrier()

    def body(i, carry):
        base = wid * EW + i * CH
        pltpu.sync_copy(src_hbm.at[pl.ds(base, CH)], src_v)
        pltpu.sync_copy(dst_hbm.at[pl.ds(base, CH)], dst_v)
        pltpu.async_copy(h_hbm.at[src_v], rows_v, sem).wait()
        pltpu.sync_copy(rows_v, acc_sh.at[dst_v], add=True)
        return carry

    lax.fori_loop(0, NCH, body, 0)
    plsc.subcore_barrier()
    pltpu.sync_copy(acc_sh.at[pl.ds(s * RW, RW)], out_hbm.at[c, pl.ds(s * RW, RW)])


def _tc1(x_ref, w1_ref, degpt_ref, hs_ref, dis_ref):
    deg = degpt_ref[:, 0:1] + degpt_ref[:, 1:2] + 1.0
    dis = lax.rsqrt(deg[:N, :])
    dis_ref[...] = dis
    h = jnp.dot(x_ref[...], w1_ref[...], preferred_element_type=jnp.float32)
    hs_ref[...] = h * dis


def _tc2(p_ref, hs1_ref, dis_ref, b1_ref, w2_ref, hs2_ref):
    agg = p_ref[0] + p_ref[1]
    agg = agg[:N, :] + hs1_ref[...]
    dis = dis_ref[...]
    z = jnp.maximum(agg * dis + b1_ref[...], 0.0)
    hs2_ref[...] = jnp.dot(z, w2_ref[...], preferred_element_type=jnp.float32) * dis


def _tc3(p_ref, hs2_ref, dis_ref, b2_ref, batch_ref, wfc_ref, bfc_ref, out_ref):
    agg = p_ref[0] + p_ref[1]
    agg = agg[:N, :] + hs2_ref[...]
    z = jnp.maximum(agg * dis_ref[...] + b2_ref[...], 0.0)
    sel = (batch_ref[...] == lax.broadcasted_iota(jnp.int32, (G, 1), 0))
    sel = sel.astype(jnp.float32)
    sums = jnp.dot(sel, z, preferred_element_type=jnp.float32)
    counts = jnp.sum(sel, axis=1, keepdims=True)
    pooled = sums / jnp.maximum(counts, 1.0)
    logits = jnp.dot(pooled, wfc_ref[...], preferred_element_type=jnp.float32)
    out_ref[...] = jax.nn.sigmoid(logits + bfc_ref[...])


def kernel(x, edge_index, batch, W1, b1, W2, b2, Wfc, bfc):
    src = edge_index[0]
    dst = edge_index[1]

    degp = _sc_degree(dst)
    degpt = degp.T

    hs1, dis = pl.pallas_call(
        _tc1,
        out_shape=[jax.ShapeDtypeStruct((N, D), jnp.float32),
                   jax.ShapeDtypeStruct((N, 1), jnp.float32)],
    )(x, W1, degpt)

    zeros_nd = jnp.zeros((NP, D), jnp.float32)
    p1 = _sc_aggregate(hs1, src, dst, zeros_nd)

    hs2 = pl.pallas_call(
        _tc2,
        out_shape=jax.ShapeDtypeStruct((N, D), jnp.float32),
    )(p1, hs1, dis, b1, W2)

    p2 = _sc_aggregate(hs2, src, dst, zeros_nd)

    out = pl.pallas_call(
        _tc3,
        out_shape=jax.ShapeDtypeStruct((G, 1), jnp.float32),
    )(p2, hs2, dis, b2, batch[None], Wfc, bfc)
    return out
